# baseline (speedup 1.0000x reference)
import jax
import jax.numpy as jnp
from jax import lax
from jax.experimental import pallas as pl
from jax.experimental.pallas import tpu as pltpu
from jax.experimental.pallas import tpu_sc as plsc

B = 4096
HIST = 20
D = 64
G = 3 * D
V = 1000000

BB = 512
NB = B // BB
NCOPY = 4
ROWS_CH = V // NCOPY

NC, NS = 2, 16
NW = NC * NS
GN = B * HIST
G_PER_W = GN // NW
G_CH = G_PER_W // 2
S_PER_W = B // NW


def _sc_gather_body(item_t, user_t, u_idx, i_idx, g_u, g_i,
                    idx_v, rows_v, sem):
    wid = lax.axis_index("s") * NC + lax.axis_index("c")
    for table, idx_hbm, out_hbm in ((item_t, u_idx, g_u), (user_t, i_idx, g_i)):
        for c in range(G_PER_W // G_CH):
            base = wid * G_PER_W + c * G_CH
            pltpu.sync_copy(idx_hbm.at[pl.ds(base, G_CH)], idx_v)
            pltpu.async_copy(table.at[idx_v], rows_v, sem).wait()
            pltpu.sync_copy(rows_v, out_hbm.at[pl.ds(base, G_CH)])


import functools as _ft


@_ft.cache
def _make_sc_gather():
  return pl.kernel(
    _sc_gather_body,
    out_type=[jax.ShapeDtypeStruct((GN, D), jnp.float32)] * 2,
    mesh=plsc.VectorSubcoreMesh(core_axis_name="c", subcore_axis_name="s", num_cores=NC, num_subcores=NS),
    compiler_params=pltpu.CompilerParams(use_tc_tiling_on_sc=False),
    scratch_types=[
        pltpu.VMEM((G_CH,), jnp.int32),
        pltpu.VMEM((G_CH, D), jnp.float32),
        pltpu.SemaphoreType.DMA,
    ],
  )


def _tc_body(w_e, wd_c, bih_c, w_hh, bhh_c, gu, gi_, du, di, um, im,
             ue_out, ie_out, num, nim,
             x_ref, hu_ref, hi_ref, ulog_ref, copy_sem):
    blk = pl.program_id(0)

    def _copies():
        for t_in, t_out in ((um, num), (im, nim)):
            for c in range(NCOPY):
                yield pltpu.make_async_copy(
                    t_in.at[pl.ds(c * ROWS_CH, ROWS_CH)],
                    t_out.at[pl.ds(c * ROWS_CH, ROWS_CH)],
                    copy_sem)

    @pl.when(blk == 0)
    def _():
        for cp in _copies():
            cp.start()

    we = w_e[...]
    whh = w_hh[...]
    b_i = bih_c[...]
    b_h = bhh_c[...]
    wdv = wd_c[...]

    nt = (((1,), (1,)), ((), ()))
    for e, g, dl in ((0, gu, du), (1, gi_, di)):
        emb = g[...].reshape(HIST * BB, D)
        giT = lax.dot_general(we, emb, nt, preferred_element_type=jnp.float32)
        giT = giT + wdv * dl[...].reshape(1, HIST * BB) + b_i
        x_ref[e] = giT

    nn = (((1,), (0,)), ((), ()))

    def step(t, HT):
        xu = x_ref[0, :, pl.ds(t * BB, BB)]
        xi = x_ref[1, :, pl.ds(t * BB, BB)]
        xt = jnp.concatenate([xu, xi], axis=1)
        gh = lax.dot_general(whh, HT, nn, preferred_element_type=jnp.float32) + b_h
        r = jax.nn.sigmoid(xt[:D] + gh[:D])
        z = jax.nn.sigmoid(xt[D:2 * D] + gh[D:2 * D])
        n = jnp.tanh(xt[2 * D:] + r * gh[2 * D:])
        HTn = (1.0 - z) * n + z * HT
        hu_ref[t] = HTn[:, :BB]
        hi_ref[t] = HTn[:, BB:]
        return HTn

    lax.fori_loop(0, HIST, step, jnp.zeros((D, 2 * BB), jnp.float32))

    def att_h(h, i_acc):
        u_h = hu_ref[h]
        ths = []
        for k in range(HIST):
            ths.append(jnp.tanh(jnp.sum(u_h * hi_ref[k], axis=0)))
        th = jnp.stack(ths, axis=0)
        ulog_ref[h] = jnp.mean(th, axis=0)
        return i_acc + th

    i_sum = lax.fori_loop(0, HIST, att_h, jnp.zeros((HIST, BB), jnp.float32))

    def soft(x):
        m = jnp.max(x, axis=0, keepdims=True)
        e = jnp.exp(x - m)
        return e / jnp.sum(e, axis=0, keepdims=True)

    u_att = soft(ulog_ref[...])
    i_att = soft(i_sum / HIST)
    ueT = jnp.zeros((D, BB), jnp.float32)
    ieT = jnp.zeros((D, BB), jnp.float32)
    for h in range(HIST):
        ueT = ueT + u_att[h][None, :] * hu_ref[h]
        ieT = ieT + i_att[h][None, :] * hi_ref[h]
    ue_out[...] = ueT.T
    ie_out[...] = ieT.T

    @pl.when(blk == NB - 1)
    def _():
        for cp in _copies():
            cp.wait()


def _tc_main(w_e, wd_c, bih_c, w_hh, bhh_c, g_u, g_i, du, di, um, im):
    full = lambda s: pl.BlockSpec(s, lambda i: (0,) * len(s))
    anyspec = pl.BlockSpec(memory_space=pl.ANY)
    return pl.pallas_call(
        _tc_body,
        grid=(NB,),
        in_specs=[
            full((G, D)), full((G, 1)), full((G, 1)), full((G, D)), full((G, 1)),
            pl.BlockSpec((HIST, BB, D), lambda i: (0, i, 0)),
            pl.BlockSpec((HIST, BB, D), lambda i: (0, i, 0)),
            pl.BlockSpec((1, 1, HIST * BB), lambda i: (i, 0, 0)),
            pl.BlockSpec((1, 1, HIST * BB), lambda i: (i, 0, 0)),
            anyspec, anyspec,
        ],
        out_specs=[pl.BlockSpec((BB, D), lambda i: (i, 0))] * 2 + [anyspec] * 2,
        out_shape=[jax.ShapeDtypeStruct((B, D), jnp.float32)] * 2
                  + [jax.ShapeDtypeStruct((V, D), jnp.float32)] * 2,
        scratch_shapes=[
            pltpu.VMEM((2, G, HIST * BB), jnp.float32),
            pltpu.VMEM((HIST, D, BB), jnp.float32),
            pltpu.VMEM((HIST, D, BB), jnp.float32),
            pltpu.VMEM((HIST, BB), jnp.float32),
            pltpu.SemaphoreType.DMA,
        ],
    )(w_e, wd_c, bih_c, w_hh, bhh_c, g_u, g_i, du, di, um, im)


def _sc_scatter_body(ue, ie, uid, iid, usrc, isrc, num_ref, nim_ref,
                     idx_v, src_v, rows_v, sem):
    wid = lax.axis_index("s") * NC + lax.axis_index("c")
    base = wid * S_PER_W
    for emb, ids, srcs, mem in ((ue, uid, usrc, num_ref), (ie, iid, isrc, nim_ref)):
        pltpu.sync_copy(ids.at[pl.ds(base, S_PER_W)], idx_v)
        pltpu.sync_copy(srcs.at[pl.ds(base, S_PER_W)], src_v)
        pltpu.async_copy(emb.at[src_v], rows_v, sem).wait()
        pltpu.async_copy(rows_v, mem.at[idx_v], sem).wait()


@_ft.cache
def _make_sc_scatter():
  return pl.kernel(
    _sc_scatter_body,
    out_type=(),
    mesh=plsc.VectorSubcoreMesh(core_axis_name="c", subcore_axis_name="s", num_cores=NC, num_subcores=NS),
    compiler_params=pltpu.CompilerParams(use_tc_tiling_on_sc=False),
    scratch_types=[
        pltpu.VMEM((S_PER_W,), jnp.int32),
        pltpu.VMEM((S_PER_W,), jnp.int32),
        pltpu.VMEM((S_PER_W, D), jnp.float32),
        pltpu.SemaphoreType.DMA,
    ],
  )


def _winner(ids):
    pos = jnp.arange(B, dtype=jnp.int32)
    eq = ids[None, :] == ids[:, None]
    return jnp.max(jnp.where(eq, pos[None, :], -1), axis=1).astype(jnp.int32)


def kernel(user_ids, user_features, item_ids, item_features,
           user_table, item_table, W_ih, W_hh, b_ih, b_hh,
           user_memory, item_memory):
    w_e = W_ih[:, :D]
    wd_c = W_ih[:, D].reshape(G, 1)
    bih_c = b_ih.reshape(G, 1)
    w_hh = W_hh
    bhh_c = b_hh.reshape(G, 1)
    u_hist_ids = user_features[:, ::2].astype(jnp.int32) + 1
    i_hist_ids = item_features[:, ::2].astype(jnp.int32) + 1
    du = (user_features[:, 1::2].T.reshape(HIST, NB, BB)
          .transpose(1, 0, 2).reshape(NB, 1, HIST * BB))
    di = (item_features[:, 1::2].T.reshape(HIST, NB, BB)
          .transpose(1, 0, 2).reshape(NB, 1, HIST * BB))
    u_idx = u_hist_ids.T.reshape(GN)
    i_idx = i_hist_ids.T.reshape(GN)

    g_u, g_i = _make_sc_gather()(item_table, user_table, u_idx, i_idx)
    g_u = g_u.reshape(HIST, B, D)
    g_i = g_i.reshape(HIST, B, D)

    ue, ie, num, nim = _tc_main(w_e, wd_c, bih_c, w_hh, bhh_c,
                                g_u, g_i, du, di, user_memory, item_memory)

    usrc = _winner(user_ids.astype(jnp.int32))
    isrc = _winner(item_ids.astype(jnp.int32))
    num_ref = jax.new_ref(num)
    nim_ref = jax.new_ref(nim)
    _make_sc_scatter()(ue, ie, user_ids.astype(jnp.int32), item_ids.astype(jnp.int32),
                usrc, isrc, num_ref, nim_ref)
    return (ue, ie, num_ref[...], nim_ref[...])


# speedup vs baseline: 10.3418x; 10.3418x over previous
import jax
import jax.numpy as jnp
from jax import lax
from jax.experimental import pallas as pl
from jax.experimental.pallas import tpu as pltpu
from jax.experimental.pallas import tpu_sc as plsc

B = 4096
HIST = 20
D = 64
G = 3 * D
V = 1000000

BB = 512
NB = B // BB

NC, NS = 2, 16
NW = NC * NS
GN = B * HIST
G_PER_W = GN // NW
G_CH = G_PER_W // 2
S_PER_W = B // NW


def _sc_gather_body(item_t, user_t, u_idx, i_idx, g_u, g_i,
                    idx_v, rows_v, sem):
    wid = lax.axis_index("s") * NC + lax.axis_index("c")
    for table, idx_hbm, out_hbm in ((item_t, u_idx, g_u), (user_t, i_idx, g_i)):
        for c in range(G_PER_W // G_CH):
            base = wid * G_PER_W + c * G_CH
            pltpu.sync_copy(idx_hbm.at[pl.ds(base, G_CH)], idx_v)
            pltpu.async_copy(table.at[idx_v], rows_v, sem).wait()
            pltpu.sync_copy(rows_v, out_hbm.at[pl.ds(base, G_CH)])


import functools as _ft


@_ft.cache
def _make_sc_gather():
  return pl.kernel(
    _sc_gather_body,
    out_type=[jax.ShapeDtypeStruct((GN, D), jnp.float32)] * 2,
    mesh=plsc.VectorSubcoreMesh(core_axis_name="c", subcore_axis_name="s", num_cores=NC, num_subcores=NS),
    compiler_params=pltpu.CompilerParams(use_tc_tiling_on_sc=False),
    scratch_types=[
        pltpu.VMEM((G_CH,), jnp.int32),
        pltpu.VMEM((G_CH, D), jnp.float32),
        pltpu.SemaphoreType.DMA,
    ],
  )


def _tc_body(w_e, wd_c, bih_c, w_hh, bhh_c, gu, gi_, du, di,
             ue_out, ie_out,
             x_ref, hu_ref, hi_ref, ulog_ref):
    we = w_e[...]
    whh = w_hh[...]
    b_i = bih_c[...]
    b_h = bhh_c[...]
    wdv = wd_c[...]

    nt = (((1,), (1,)), ((), ()))
    for e, g, dl in ((0, gu, du), (1, gi_, di)):
        emb = g[...].reshape(HIST * BB, D)
        giT = lax.dot_general(we, emb, nt, preferred_element_type=jnp.float32)
        giT = giT + wdv * dl[...].reshape(1, HIST * BB) + b_i
        x_ref[e] = giT

    nn = (((1,), (0,)), ((), ()))

    def step(t, HT):
        xu = x_ref[0, :, pl.ds(t * BB, BB)]
        xi = x_ref[1, :, pl.ds(t * BB, BB)]
        xt = jnp.concatenate([xu, xi], axis=1)
        gh = lax.dot_general(whh, HT, nn, preferred_element_type=jnp.float32) + b_h
        r = jax.nn.sigmoid(xt[:D] + gh[:D])
        z = jax.nn.sigmoid(xt[D:2 * D] + gh[D:2 * D])
        n = jnp.tanh(xt[2 * D:] + r * gh[2 * D:])
        HTn = (1.0 - z) * n + z * HT
        hu_ref[t] = HTn[:, :BB]
        hi_ref[t] = HTn[:, BB:]
        return HTn

    lax.fori_loop(0, HIST, step, jnp.zeros((D, 2 * BB), jnp.float32))

    def att_h(h, i_acc):
        u_h = hu_ref[h]
        ths = []
        for k in range(HIST):
            ths.append(jnp.tanh(jnp.sum(u_h * hi_ref[k], axis=0)))
        th = jnp.stack(ths, axis=0)
        ulog_ref[h] = jnp.mean(th, axis=0)
        return i_acc + th

    i_sum = lax.fori_loop(0, HIST, att_h, jnp.zeros((HIST, BB), jnp.float32))

    def soft(x):
        m = jnp.max(x, axis=0, keepdims=True)
        e = jnp.exp(x - m)
        return e / jnp.sum(e, axis=0, keepdims=True)

    u_att = soft(ulog_ref[...])
    i_att = soft(i_sum / HIST)
    ueT = jnp.zeros((D, BB), jnp.float32)
    ieT = jnp.zeros((D, BB), jnp.float32)
    for h in range(HIST):
        ueT = ueT + u_att[h][None, :] * hu_ref[h]
        ieT = ieT + i_att[h][None, :] * hi_ref[h]
    ue_out[...] = ueT.T
    ie_out[...] = ieT.T


def _tc_main(w_e, wd_c, bih_c, w_hh, bhh_c, g_u, g_i, du, di):
    full = lambda s: pl.BlockSpec(s, lambda i: (0,) * len(s))
    return pl.pallas_call(
        _tc_body,
        grid=(NB,),
        in_specs=[
            full((G, D)), full((G, 1)), full((G, 1)), full((G, D)), full((G, 1)),
            pl.BlockSpec((HIST, BB, D), lambda i: (0, i, 0)),
            pl.BlockSpec((HIST, BB, D), lambda i: (0, i, 0)),
            pl.BlockSpec((1, 1, HIST * BB), lambda i: (i, 0, 0)),
            pl.BlockSpec((1, 1, HIST * BB), lambda i: (i, 0, 0)),
        ],
        out_specs=[pl.BlockSpec((BB, D), lambda i: (i, 0))] * 2,
        out_shape=[jax.ShapeDtypeStruct((B, D), jnp.float32)] * 2,
        scratch_shapes=[
            pltpu.VMEM((2, G, HIST * BB), jnp.float32),
            pltpu.VMEM((HIST, D, BB), jnp.float32),
            pltpu.VMEM((HIST, D, BB), jnp.float32),
            pltpu.VMEM((HIST, BB), jnp.float32),
        ],
    )(w_e, wd_c, bih_c, w_hh, bhh_c, g_u, g_i, du, di)


R_CH = 20000
NCH = V // R_CH


def _merge_body(ids_s, perm, bounds, mem, emb, out):
    c = pl.program_id(0)
    out[...] = mem[...]
    base = c * R_CH

    def patch(j, _):
        r = ids_s[j] - base
        srow = perm[j]
        out[pl.ds(r, 1), :] = emb[pl.ds(srow, 1), :]
        return 0

    lax.fori_loop(bounds[c], bounds[c + 1], patch, 0)


def _merge_copy(mem, emb, ids_s, perm, bounds):
    smem = pl.BlockSpec(memory_space=pltpu.SMEM)
    return pl.pallas_call(
        _merge_body,
        grid=(NCH,),
        in_specs=[
            smem, smem, smem,
            pl.BlockSpec((R_CH, D), lambda i: (i, 0)),
            pl.BlockSpec((B, D), lambda i: (0, 0)),
        ],
        out_specs=pl.BlockSpec((R_CH, D), lambda i: (i, 0)),
        out_shape=jax.ShapeDtypeStruct((V, D), jnp.float32),
    )(ids_s, perm, bounds, mem, emb)


def kernel(user_ids, user_features, item_ids, item_features,
           user_table, item_table, W_ih, W_hh, b_ih, b_hh,
           user_memory, item_memory):
    w_e = W_ih[:, :D]
    wd_c = W_ih[:, D].reshape(G, 1)
    bih_c = b_ih.reshape(G, 1)
    w_hh = W_hh
    bhh_c = b_hh.reshape(G, 1)
    u_hist_ids = user_features[:, ::2].astype(jnp.int32) + 1
    i_hist_ids = item_features[:, ::2].astype(jnp.int32) + 1
    du = (user_features[:, 1::2].T.reshape(HIST, NB, BB)
          .transpose(1, 0, 2).reshape(NB, 1, HIST * BB))
    di = (item_features[:, 1::2].T.reshape(HIST, NB, BB)
          .transpose(1, 0, 2).reshape(NB, 1, HIST * BB))
    u_idx = u_hist_ids.T.reshape(GN)
    i_idx = i_hist_ids.T.reshape(GN)

    g_u, g_i = _make_sc_gather()(item_table, user_table, u_idx, i_idx)
    g_u = g_u.reshape(HIST, B, D)
    g_i = g_i.reshape(HIST, B, D)

    ue, ie = _tc_main(w_e, wd_c, bih_c, w_hh, bhh_c, g_u, g_i, du, di)

    grid_edges = jnp.arange(0, V + 1, R_CH, dtype=jnp.int32)
    outs = []
    for ids, emb, mem in ((user_ids, ue, user_memory), (item_ids, ie, item_memory)):
        ids = ids.astype(jnp.int32)
        perm = jnp.argsort(ids, stable=True).astype(jnp.int32)
        ids_s = ids[perm]
        bounds = jnp.searchsorted(ids_s, grid_edges).astype(jnp.int32)
        outs.append(_merge_copy(mem, emb, ids_s, perm, bounds))
    return (ue, ie, outs[0], outs[1])
